# baseline (speedup 1.0000x reference)
import functools

import jax
import jax.numpy as jnp
from jax import lax
from jax.experimental import pallas as pl
from jax.experimental.pallas import tpu as pltpu
from jax.experimental.pallas import tpu_sc as plsc

_N = 10000
_E = 320000
_D = 128
_NC = 2
_NS = 16
_NW = _NC * _NS
_EW = _E // _NW
_K = 80
_CH = _EW // _K
_RT = 632
_NP = _NS * _RT
_CW = 8
_ZR = 128


@functools.cache
def _build_sc():
    mesh = plsc.VectorSubcoreMesh(core_axis_name="c", subcore_axis_name="s")
    return functools.partial(
        pl.kernel,
        out_type=(jax.ShapeDtypeStruct((_NC, _NP, _D), jnp.float32),
                  jax.ShapeDtypeStruct((_NC, _NP, _CW), jnp.float32)),
        mesh=mesh,
        scratch_types=[
            pltpu.VMEM_SHARED((_NP, _D), jnp.float32),
            pltpu.VMEM_SHARED((_NP, _CW), jnp.float32),
            pltpu.VMEM((_CH, _K), jnp.int32),
            pltpu.VMEM((_CH, _K), jnp.int32),
            pltpu.VMEM((_K, _D), jnp.float32),
            pltpu.VMEM((_K, _D), jnp.float32),
            pltpu.VMEM((_K, _CW), jnp.float32),
            pltpu.SemaphoreType.DMA,
            pltpu.SemaphoreType.DMA,
            pltpu.SemaphoreType.DMA,
            pltpu.SemaphoreType.DMA,
        ],
        compiler_params=pltpu.CompilerParams(use_tc_tiling_on_sc=False),
    )(_sc_scatter)


def _sc_scatter(x, src3, dst3, zeros, zc, ones, acc_out, cnt_out,
                acc_sh, cnt_sh, src_a, dst_a, b0, b1, ones_v, g0, g1, c0, c1):
    cid = lax.axis_index("c")
    sid = lax.axis_index("s")
    wid = sid * _NC + cid
    r0 = sid * _RT

    for t in range(_RT // _ZR):
        pltpu.sync_copy(zeros, acc_sh.at[pl.ds(r0 + t * _ZR, _ZR)])
        pltpu.sync_copy(zc, cnt_sh.at[pl.ds(r0 + t * _ZR, _ZR)])
    rem = _RT % _ZR
    pltpu.sync_copy(zeros.at[pl.ds(0, rem)],
                    acc_sh.at[pl.ds(r0 + _RT - rem, rem)])
    pltpu.sync_copy(zc.at[pl.ds(0, rem)],
                    cnt_sh.at[pl.ds(r0 + _RT - rem, rem)])
    pltpu.sync_copy(src3.at[wid], src_a)
    pltpu.sync_copy(dst3.at[wid], dst_a)
    pltpu.sync_copy(ones, ones_v)
    plsc.subcore_barrier()

    def start(i, buf, gsem):
        pltpu.async_copy(x.at[src_a.at[i]], buf, gsem)

    def fin(i, buf, gsem, csem):
        pltpu.make_async_copy(x.at[src_a.at[i]], buf, gsem).wait()
        pltpu.sync_copy(buf, acc_sh.at[dst_a.at[i]], add=True)

        @pl.when(i >= 2)
        def _():
            pltpu.make_async_copy(ones_v, cnt_sh.at[dst_a.at[i]], csem).wait()

        pltpu.async_copy(ones_v, cnt_sh.at[dst_a.at[i]], csem, add=True)

    start(0, b0, g0)

    def body(j, carry):
        i = 2 * j
        start(i + 1, b1, g1)
        fin(i, b0, g0, c0)
        start(i + 2, b0, g0)
        fin(i + 1, b1, g1, c1)
        return carry

    lax.fori_loop(0, (_CH - 1) // 2, body, 0)
    fin(_CH - 1, b0, g0, c0)
    pltpu.make_async_copy(ones_v, cnt_sh.at[dst_a.at[0]], c1).wait()
    pltpu.make_async_copy(ones_v, cnt_sh.at[dst_a.at[0]], c0).wait()

    plsc.subcore_barrier()
    pltpu.sync_copy(acc_sh.at[pl.ds(r0, _RT)],
                    acc_out.at[cid, pl.ds(r0, _RT)])
    pltpu.sync_copy(cnt_sh.at[pl.ds(r0, _RT)],
                    cnt_out.at[cid, pl.ds(r0, _RT)])


_BN = 1000


def _tc_z_body(x_ref, wr_ref, b_ref, o_ref):
    o_ref[...] = (
        jnp.dot(x_ref[...], wr_ref[...], preferred_element_type=jnp.float32)
        + b_ref[...]
    )


def _tc_z(x, wr, b):
    return pl.pallas_call(
        _tc_z_body,
        grid=(_N // _BN,),
        in_specs=[
            pl.BlockSpec((_BN, _D), lambda i: (i, 0)),
            pl.BlockSpec((_D, _D), lambda i: (0, 0)),
            pl.BlockSpec((1, _D), lambda i: (0, 0)),
        ],
        out_specs=pl.BlockSpec((_BN, _D), lambda i: (i, 0)),
        out_shape=jax.ShapeDtypeStruct((_N, _D), jnp.float32),
    )(x, wr, b)


def _tc_body(p_ref, cnt_ref, z_ref, wl_ref, o_ref):
    s = p_ref[0] + p_ref[1]
    c = jnp.maximum(cnt_ref[0, :, 0:1] + cnt_ref[1, :, 0:1], 1.0)
    mean = s / c
    o_ref[...] = (
        jnp.dot(mean, wl_ref[...], preferred_element_type=jnp.float32)
        + z_ref[...]
    )


def _tc_combine(p, cnt, z, wl):
    return pl.pallas_call(
        _tc_body,
        grid=(_N // _BN,),
        in_specs=[
            pl.BlockSpec((_NC, _BN, _D), lambda i: (0, i, 0)),
            pl.BlockSpec((_NC, _BN, _CW), lambda i: (0, i, 0)),
            pl.BlockSpec((_BN, _D), lambda i: (i, 0)),
            pl.BlockSpec((_D, _D), lambda i: (0, 0)),
        ],
        out_specs=pl.BlockSpec((_BN, _D), lambda i: (i, 0)),
        out_shape=jax.ShapeDtypeStruct((_N, _D), jnp.float32),
    )(p, cnt, z, wl)


def kernel(x, edge_index, W1_l, b1_l, W1_r, W2_l, b2_l, W2_r):
    src3 = edge_index[0].reshape(_NW, _CH, _K)
    dst3 = edge_index[1].reshape(_NW, _CH, _K)
    zeros = jnp.zeros((_ZR, _D), jnp.float32)
    zc = jnp.zeros((_ZR, _CW), jnp.float32)
    ones = jnp.ones((_K, _CW), jnp.float32)
    acc, cnt = _build_sc()(x, src3, dst3, zeros, zc, ones)
    z = _tc_z(x, W2_r, b2_l.reshape(1, _D))
    return _tc_combine(acc, cnt, z, W2_l)


# speedup vs baseline: 1.0298x; 1.0298x over previous
import functools

import jax
import jax.numpy as jnp
from jax import lax
from jax.experimental import pallas as pl
from jax.experimental.pallas import tpu as pltpu
from jax.experimental.pallas import tpu_sc as plsc

_N = 10000
_E = 320000
_D = 128
_NC = 2
_NS = 16
_NW = _NC * _NS
_EW = _E // _NW
_K = 80
_CH = _EW // _K
_RT = 632
_NP = _NS * _RT
_CW = 8
_ZR = 128


@functools.cache
def _build_sc():
    mesh = plsc.VectorSubcoreMesh(core_axis_name="c", subcore_axis_name="s")
    return functools.partial(
        pl.kernel,
        out_type=(jax.ShapeDtypeStruct((_NC, _NP, _D), jnp.float32),
                  jax.ShapeDtypeStruct((_NC, _NP, _CW), jnp.float32)),
        mesh=mesh,
        scratch_types=[
            pltpu.VMEM_SHARED((_NP, _D), jnp.float32),
            pltpu.VMEM_SHARED((_NP, _CW), jnp.float32),
            pltpu.VMEM((_CH, _K), jnp.int32),
            pltpu.VMEM((_CH, _K), jnp.int32),
            pltpu.VMEM((_K, _D), jnp.float32),
            pltpu.VMEM((_K, _D), jnp.float32),
            pltpu.VMEM((_K, _CW), jnp.float32),
            pltpu.SemaphoreType.DMA,
            pltpu.SemaphoreType.DMA,
            pltpu.SemaphoreType.DMA,
            pltpu.SemaphoreType.DMA,
        ],
        compiler_params=pltpu.CompilerParams(use_tc_tiling_on_sc=False),
    )(_sc_scatter)


def _sc_scatter(x, src3, dst3, zeros, zc, ones, acc_out, cnt_out,
                acc_sh, cnt_sh, src_a, dst_a, b0, b1, ones_v, g0, g1, c0, c1):
    cid = lax.axis_index("c")
    sid = lax.axis_index("s")
    wid = sid * _NC + cid
    r0 = sid * _RT

    rem = _RT % _ZR
    sems = (g0, g1, c0, c1)
    pend = []
    for t in range(_RT // _ZR):
        pend.append(pltpu.async_copy(
            zeros, acc_sh.at[pl.ds(r0 + t * _ZR, _ZR)], sems[t % 4]))
        pend.append(pltpu.async_copy(
            zc, cnt_sh.at[pl.ds(r0 + t * _ZR, _ZR)], sems[(t + 1) % 4]))
    pend.append(pltpu.async_copy(
        zeros.at[pl.ds(0, rem)],
        acc_sh.at[pl.ds(r0 + _RT - rem, rem)], g0))
    pend.append(pltpu.async_copy(
        zc.at[pl.ds(0, rem)],
        cnt_sh.at[pl.ds(r0 + _RT - rem, rem)], g1))
    pend.append(pltpu.async_copy(src3.at[wid], src_a, c0))
    pend.append(pltpu.async_copy(dst3.at[wid], dst_a, c1))
    pend.append(pltpu.async_copy(ones, ones_v, g0))
    for dsc in pend:
        dsc.wait()
    plsc.subcore_barrier()

    def start(i, buf, gsem):
        pltpu.async_copy(x.at[src_a.at[i]], buf, gsem)

    def fin(i, buf, gsem, csem):
        pltpu.make_async_copy(x.at[src_a.at[i]], buf, gsem).wait()
        pltpu.sync_copy(buf, acc_sh.at[dst_a.at[i]], add=True)

        @pl.when(i >= 2)
        def _():
            pltpu.make_async_copy(ones_v, cnt_sh.at[dst_a.at[i]], csem).wait()

        pltpu.async_copy(ones_v, cnt_sh.at[dst_a.at[i]], csem, add=True)

    start(0, b0, g0)

    def body(j, carry):
        i = 2 * j
        start(i + 1, b1, g1)
        fin(i, b0, g0, c0)
        start(i + 2, b0, g0)
        fin(i + 1, b1, g1, c1)
        return carry

    lax.fori_loop(0, (_CH - 1) // 2, body, 0)
    fin(_CH - 1, b0, g0, c0)
    pltpu.make_async_copy(ones_v, cnt_sh.at[dst_a.at[0]], c1).wait()
    pltpu.make_async_copy(ones_v, cnt_sh.at[dst_a.at[0]], c0).wait()

    plsc.subcore_barrier()
    pltpu.sync_copy(acc_sh.at[pl.ds(r0, _RT)],
                    acc_out.at[cid, pl.ds(r0, _RT)])
    pltpu.sync_copy(cnt_sh.at[pl.ds(r0, _RT)],
                    cnt_out.at[cid, pl.ds(r0, _RT)])


_BN = 1000


def _tc_body(p_ref, cnt_ref, x_ref, wl_ref, wr_ref, b_ref, o_ref):
    s = p_ref[0] + p_ref[1]
    c = jnp.maximum(cnt_ref[0, :, 0:1] + cnt_ref[1, :, 0:1], 1.0)
    mean = s / c
    o_ref[...] = (
        jnp.dot(mean, wl_ref[...], preferred_element_type=jnp.float32)
        + jnp.dot(x_ref[...], wr_ref[...], preferred_element_type=jnp.float32)
        + b_ref[...]
    )


def _tc_combine(p, cnt, x, wl, wr, b):
    return pl.pallas_call(
        _tc_body,
        grid=(_N // _BN,),
        in_specs=[
            pl.BlockSpec((_NC, _BN, _D), lambda i: (0, i, 0)),
            pl.BlockSpec((_NC, _BN, _CW), lambda i: (0, i, 0)),
            pl.BlockSpec((_BN, _D), lambda i: (i, 0)),
            pl.BlockSpec((_D, _D), lambda i: (0, 0)),
            pl.BlockSpec((_D, _D), lambda i: (0, 0)),
            pl.BlockSpec((1, _D), lambda i: (0, 0)),
        ],
        out_specs=pl.BlockSpec((_BN, _D), lambda i: (i, 0)),
        out_shape=jax.ShapeDtypeStruct((_N, _D), jnp.float32),
    )(p, cnt, x, wl, wr, b)


def kernel(x, edge_index, W1_l, b1_l, W1_r, W2_l, b2_l, W2_r):
    src3 = edge_index[0].reshape(_NW, _CH, _K)
    dst3 = edge_index[1].reshape(_NW, _CH, _K)
    zeros = jnp.zeros((_ZR, _D), jnp.float32)
    zc = jnp.zeros((_ZR, _CW), jnp.float32)
    ones = jnp.ones((_K, _CW), jnp.float32)
    acc, cnt = _build_sc()(x, src3, dst3, zeros, zc, ones)
    return _tc_combine(acc, cnt, x, W2_l, W2_r, b2_l.reshape(1, _D))
